# Initial kernel scaffold; baseline (speedup 1.0000x reference)
import jax
import jax.numpy as jnp
from jax.experimental import pallas as pl


def kernel(hidden_states, Wg, W_in, W_out):
    raise NotImplementedError("write your pallas kernel here")


# speedup vs baseline: 3.3070x; 3.3070x over previous
import functools

import jax
import jax.numpy as jnp
from jax import lax
from jax.experimental import pallas as pl
from jax.experimental.pallas import tpu as pltpu
from jax.experimental.pallas import tpu_sc as plsc

D = 768
FF = 1536
E = 8
K = 2
T = 2048
C = 640
TRASH = E * C
BUF_ROWS = E * C + 8
FB = 512
NJ = FF // FB
TB = 256
NB = T // TB


def _router_body(x_ref, wg_ref, sd_ref, sc_ref, w_ref,
                 oh1_ref, oh2_ref, c1_ref, c2_ref):
    xf = x_ref[...]
    logits = jnp.dot(xf, wg_ref[...], preferred_element_type=jnp.float32)
    probs = jax.nn.softmax(logits, axis=-1)
    iota = lax.broadcasted_iota(jnp.int32, (T, E), 1)
    m1 = jnp.max(probs, axis=-1, keepdims=True)
    i1 = jnp.min(jnp.where(probs >= m1, iota, E), axis=-1, keepdims=True)
    oh1 = iota == i1
    probsm = jnp.where(oh1, -jnp.inf, probs)
    m2 = jnp.max(probsm, axis=-1, keepdims=True)
    i2 = jnp.min(jnp.where(probsm >= m2, iota, E), axis=-1, keepdims=True)
    oh2 = iota == i2
    denom = m1 + m2 + 1e-9
    g1 = m1 / denom
    g2 = m2 / denom

    oh1_ref[...] = oh1.astype(jnp.float32)
    oh2_ref[...] = oh2.astype(jnp.float32)
    rows = lax.broadcasted_iota(jnp.int32, (TB, TB), 0)
    cols = lax.broadcasted_iota(jnp.int32, (TB, TB), 1)
    tri = (rows >= cols).astype(jnp.float32)

    def body(b, carries):
        car1, car2 = carries
        blk1 = oh1_ref[pl.ds(b * TB, TB), :]
        blk2 = oh2_ref[pl.ds(b * TB, TB), :]
        cs1 = jnp.dot(tri, blk1, preferred_element_type=jnp.float32) + car1
        cs2 = jnp.dot(tri, blk2, preferred_element_type=jnp.float32) + car2
        c1_ref[pl.ds(b * TB, TB), :] = jnp.sum(cs1 * blk1, axis=-1, keepdims=True)
        c2_ref[pl.ds(b * TB, TB), :] = jnp.sum(cs2 * blk2, axis=-1, keepdims=True)
        return (car1 + jnp.sum(blk1, axis=0, keepdims=True),
                car2 + jnp.sum(blk2, axis=0, keepdims=True))

    zero8 = jnp.zeros((1, E), jnp.float32)
    tot1, _ = lax.fori_loop(0, NB, body, (zero8, zero8))

    pos0 = c1_ref[...].astype(jnp.int32) - 1
    tot1_own = jnp.sum(tot1 * oh2_ref[...], axis=-1, keepdims=True)
    pos1 = (c2_ref[...] + tot1_own).astype(jnp.int32) - 1
    keep0 = pos0 < C
    keep1 = pos1 < C
    slotc0 = i1 * C + jnp.where(keep0, pos0, 0)
    slotc1 = i2 * C + jnp.where(keep1, pos1, 0)
    sd_ref[:, 0:1] = jnp.where(keep0, slotc0, TRASH)
    sd_ref[:, 1:2] = jnp.where(keep1, slotc1, TRASH)
    sc_ref[:, 0:1] = slotc0
    sc_ref[:, 1:2] = slotc1
    w_ref[:, 0:1] = jnp.where(keep0, g1, 0.0)
    w_ref[:, 1:2] = jnp.where(keep1, g2, 0.0)


def _router(xf, Wg):
    return pl.pallas_call(
        _router_body,
        out_shape=[jax.ShapeDtypeStruct((T, K), jnp.int32),
                   jax.ShapeDtypeStruct((T, K), jnp.int32),
                   jax.ShapeDtypeStruct((T, K), jnp.float32)],
        scratch_shapes=[pltpu.VMEM((T, E), jnp.float32),
                        pltpu.VMEM((T, E), jnp.float32),
                        pltpu.VMEM((T, 1), jnp.float32),
                        pltpu.VMEM((T, 1), jnp.float32)],
    )(xf, Wg)


def _dispatch(xf, slots_flat):
    info = plsc.get_sparse_core_info()
    nc, ns = info.num_cores, info.num_subcores
    nw = nc * ns
    ch = (K * T) // nw
    mesh = plsc.VectorSubcoreMesh(core_axis_name="c", subcore_axis_name="s")

    @functools.partial(
        pl.kernel, mesh=mesh,
        out_type=jax.ShapeDtypeStruct((BUF_ROWS, D), jnp.float32),
        scratch_types=[pltpu.VMEM((ch,), jnp.int32),
                       pltpu.VMEM((ch, D), jnp.float32),
                       pltpu.SemaphoreType.DMA],
    )
    def k(x_hbm, sl_hbm, buf_hbm, idx_v, rows_v, sem):
        wid = lax.axis_index("s") * nc + lax.axis_index("c")
        base = wid * ch
        tok = lax.rem(base, T)
        pltpu.sync_copy(sl_hbm.at[pl.ds(base, ch)], idx_v)
        pltpu.sync_copy(x_hbm.at[pl.ds(tok, ch)], rows_v)
        pltpu.async_copy(rows_v, buf_hbm.at[idx_v], sem).wait()

    return k(xf, slots_flat)


def _ffn_body(x_ref, wa_ref, wg_ref, wo_ref, o_ref):
    j = pl.program_id(1)
    a = jnp.dot(x_ref[...], wa_ref[0], preferred_element_type=jnp.float32)
    g = jnp.dot(x_ref[...], wg_ref[0], preferred_element_type=jnp.float32)
    act = jax.nn.gelu(a) * g
    part = jnp.dot(act, wo_ref[0], preferred_element_type=jnp.float32)

    @pl.when(j == 0)
    def _():
        o_ref[...] = part

    @pl.when(j > 0)
    def _():
        o_ref[...] += part


def _ffn(buf, W_in, W_out):
    return pl.pallas_call(
        _ffn_body,
        grid=(E, NJ),
        in_specs=[
            pl.BlockSpec((C, D), lambda e, j: (e, 0)),
            pl.BlockSpec((1, D, FB), lambda e, j: (e, 0, j)),
            pl.BlockSpec((1, D, FB), lambda e, j: (e, 0, j + NJ)),
            pl.BlockSpec((1, FB, D), lambda e, j: (e, j, 0)),
        ],
        out_specs=pl.BlockSpec((C, D), lambda e, j: (e, 0)),
        out_shape=jax.ShapeDtypeStruct((E * C, D), jnp.float32),
        compiler_params=pltpu.CompilerParams(
            dimension_semantics=("arbitrary", "arbitrary")),
    )(buf, W_in, W_in, W_out)


def _gather(out_rows, slots_flat):
    info = plsc.get_sparse_core_info()
    nc, ns = info.num_cores, info.num_subcores
    nw = nc * ns
    ch = (K * T) // nw
    mesh = plsc.VectorSubcoreMesh(core_axis_name="c", subcore_axis_name="s")

    @functools.partial(
        pl.kernel, mesh=mesh,
        out_type=jax.ShapeDtypeStruct((K * T, D), jnp.float32),
        scratch_types=[pltpu.VMEM((ch,), jnp.int32),
                       pltpu.VMEM((ch, D), jnp.float32),
                       pltpu.SemaphoreType.DMA],
    )
    def k(o_hbm, sl_hbm, g_hbm, idx_v, rows_v, sem):
        wid = lax.axis_index("s") * nc + lax.axis_index("c")
        base = wid * ch
        pltpu.sync_copy(sl_hbm.at[pl.ds(base, ch)], idx_v)
        pltpu.async_copy(o_hbm.at[idx_v], rows_v, sem).wait()
        pltpu.sync_copy(rows_v, g_hbm.at[pl.ds(base, ch)])

    return k(out_rows, slots_flat)


def _wsum_body(g_ref, w_ref, y_ref):
    y_ref[...] = (g_ref[0] * w_ref[:, 0:1] + g_ref[1] * w_ref[:, 1:2])


def _wsum(g, ws):
    cb = 128
    return pl.pallas_call(
        _wsum_body,
        grid=(D // cb,),
        in_specs=[
            pl.BlockSpec((K, T, cb), lambda j: (0, 0, j)),
            pl.BlockSpec((T, K), lambda j: (0, 0)),
        ],
        out_specs=pl.BlockSpec((T, cb), lambda j: (0, j)),
        out_shape=jax.ShapeDtypeStruct((T, D), jnp.float32),
    )(g, ws)


def kernel(hidden_states, Wg, W_in, W_out):
    B, S, Dm = hidden_states.shape
    xf = hidden_states.reshape(T, D)
    sd, sc_, ws = _router(xf, Wg)
    sdf = sd.T.reshape(-1)
    scf = sc_.T.reshape(-1)
    buf = _dispatch(xf, sdf)
    out = _ffn(buf, W_in, W_out)
    g = _gather(out, scf)
    y = _wsum(g.reshape(K, T, D), ws)
    return y.reshape(B, S, Dm)
